# Initial kernel scaffold; baseline (speedup 1.0000x reference)
import jax
import jax.numpy as jnp
from jax.experimental import pallas as pl


def kernel(x, edge_index, W_fc, W_att):
    raise NotImplementedError("write your pallas kernel here")


# speedup vs baseline: 7.1255x; 7.1255x over previous
import functools

import jax
import jax.numpy as jnp
from jax import lax
from jax.experimental import pallas as pl
from jax.experimental.pallas import tpu as pltpu
from jax.experimental.pallas import tpu_sc as plsc

N_NODES = 10000
DIM = 128
N_EDGES = 320000

NC = 2
NS = 16
L = 16
NW = NC * NS
E_PER_W = N_EDGES // NW
CHUNK = 80
N_CHUNKS = E_PER_W // CHUNK
N_PAD = 10240
ROWS_PER_TILE = N_PAD // NS
ZBLK = 128


def _tc_pre_body(x_ref, wfc_ref, watt_ref, h_ref, sl_ref, sr_ref):
    h = jnp.dot(x_ref[...], wfc_ref[...].T, preferred_element_type=jnp.float32)
    h_ref[...] = h
    al = watt_ref[0, :DIM]
    ar = watt_ref[0, DIM:]
    sl_ref[...] = jnp.sum(h * al[None, :], axis=1)
    sr_ref[...] = jnp.sum(h * ar[None, :], axis=1)


def _tc_add_body(a_ref, b_ref, o_ref):
    o_ref[...] = a_ref[...] + b_ref[...]


def _sc_edge_body(h_hbm, sl_hbm, sr_hbm, src_hbm, dst_hbm, out_hbm,
                  sl_v, sr_v, src_v, dst_v, rows_v, z_v, facc, gsem):
    c = lax.axis_index("c")
    s = lax.axis_index("s")
    wid = c * NS + s
    row0 = s * ROWS_PER_TILE

    pltpu.sync_copy(sl_hbm, sl_v)
    pltpu.sync_copy(sr_hbm, sr_v)

    zeros = jnp.zeros((L,), jnp.float32)

    @pl.loop(0, ZBLK)
    def _zfill(i):
        for g in range(DIM // L):
            z_v[i, pl.ds(g * L, L)] = zeros

    for kblk in range(ROWS_PER_TILE // ZBLK):
        pltpu.sync_copy(z_v, facc.at[pl.ds(row0 + kblk * ZBLK, ZBLK)])
    plsc.subcore_barrier()

    wbase = wid * E_PER_W

    @pl.loop(0, N_CHUNKS)
    def _edges(i):
        base = wbase + i * CHUNK
        pltpu.sync_copy(src_hbm.at[pl.ds(base, CHUNK)], src_v)
        pltpu.sync_copy(dst_hbm.at[pl.ds(base, CHUNK)], dst_v)
        pltpu.async_copy(h_hbm.at[src_v], rows_v, gsem).wait()
        for g in range(CHUNK // L):
            s16 = src_v[pl.ds(g * L, L)]
            d16 = dst_v[pl.ds(g * L, L)]
            slv = plsc.load_gather(sl_v, [s16])
            srv = plsc.load_gather(sr_v, [d16])
            e16 = 1.0 / (1.0 + jnp.exp(-(slv + srv)))
            for l in range(L):
                ce = g * L + l
                es = e16[l]
                for q in range(DIM // L):
                    rows_v[ce, pl.ds(q * L, L)] = (
                        rows_v[ce, pl.ds(q * L, L)] * es)
        pltpu.sync_copy(rows_v, facc.at[dst_v], add=True)

    plsc.subcore_barrier()
    for kblk in range(ROWS_PER_TILE // ZBLK):
        r = row0 + kblk * ZBLK
        pltpu.sync_copy(facc.at[pl.ds(r, ZBLK)], out_hbm.at[c, pl.ds(r, ZBLK)])


@functools.lru_cache(maxsize=1)
def _sc_edge_kernel():
    mesh = plsc.VectorSubcoreMesh(
        core_axis_name="c", subcore_axis_name="s",
        num_cores=NC, num_subcores=NS)
    return pl.kernel(
        _sc_edge_body,
        out_type=jax.ShapeDtypeStruct((NC, N_PAD, DIM), jnp.float32),
        mesh=mesh,
        compiler_params=pltpu.CompilerParams(needs_layout_passes=False),
        scratch_types=[
            pltpu.VMEM((N_NODES,), jnp.float32),
            pltpu.VMEM((N_NODES,), jnp.float32),
            pltpu.VMEM((CHUNK,), jnp.int32),
            pltpu.VMEM((CHUNK,), jnp.int32),
            pltpu.VMEM((CHUNK, DIM), jnp.float32),
            pltpu.VMEM((ZBLK, DIM), jnp.float32),
            pltpu.VMEM_SHARED((N_PAD, DIM), jnp.float32),
            pltpu.SemaphoreType.DMA,
        ],
    )


def kernel(x, edge_index, W_fc, W_att):
    ei = edge_index.astype(jnp.int32)
    src = ei[0]
    dst = ei[1]
    h, sl, sr = pl.pallas_call(
        _tc_pre_body,
        out_shape=[
            jax.ShapeDtypeStruct((N_NODES, DIM), jnp.float32),
            jax.ShapeDtypeStruct((N_NODES,), jnp.float32),
            jax.ShapeDtypeStruct((N_NODES,), jnp.float32),
        ],
    )(x, W_fc, W_att)
    fp = _sc_edge_kernel()(h, sl, sr, src, dst)
    f = pl.pallas_call(
        _tc_add_body,
        out_shape=jax.ShapeDtypeStruct((N_NODES, DIM), jnp.float32),
    )(fp[0, :N_NODES], fp[1, :N_NODES])
    return f


# speedup vs baseline: 9.1995x; 1.2911x over previous
import functools

import jax
import jax.numpy as jnp
from jax import lax
from jax.experimental import pallas as pl
from jax.experimental.pallas import tpu as pltpu
from jax.experimental.pallas import tpu_sc as plsc

N_NODES = 10000
DIM = 128
N_EDGES = 320000

NC = 2
NS = 16
L = 16
NW = NC * NS
E_PER_W = N_EDGES // NW
CHUNK = 40
N_CHUNKS = E_PER_W // CHUNK
NBUF = 5
N_PAD = 10240
ROWS_PER_TILE = N_PAD // NS
ZBLK = 128


def _tc_pre_body(x_ref, wfc_ref, watt_ref, h_ref, sl_ref, sr_ref):
    h = jnp.dot(x_ref[...], wfc_ref[...].T, preferred_element_type=jnp.float32)
    h_ref[...] = h
    al = watt_ref[0, :DIM]
    ar = watt_ref[0, DIM:]
    sl_ref[...] = jnp.sum(h * al[None, :], axis=1)
    sr_ref[...] = jnp.sum(h * ar[None, :], axis=1)


def _tc_add_body(a_ref, b_ref, o_ref):
    o_ref[...] = a_ref[...] + b_ref[...]


def _sc_edge_body(h_hbm, sl_hbm, sr_hbm, src_hbm, dst_hbm, out_hbm,
                  sl_v, sr_v, src_v, dst_v, e_v, rows, sem_g, sem_s, facc):
    c = lax.axis_index("c")
    s = lax.axis_index("s")
    wid = c * NS + s
    row0 = s * ROWS_PER_TILE

    pltpu.sync_copy(sl_hbm, sl_v)
    pltpu.sync_copy(sr_hbm, sr_v)

    zeros = jnp.zeros((L,), jnp.float32)

    @pl.loop(0, CHUNK)
    def _zfill(i):
        for g in range(DIM // L):
            rows[0][i, pl.ds(g * L, L)] = zeros

    for kblk in range(ROWS_PER_TILE // CHUNK):
        pltpu.sync_copy(rows[0], facc.at[pl.ds(row0 + kblk * CHUNK, CHUNK)])
    plsc.subcore_barrier()

    n_win = (CHUNK + L - 1) // L
    win_off = [min(g * L, CHUNK - L) for g in range(n_win)]

    n_super = N_CHUNKS // NBUF

    @pl.loop(0, n_super)
    def _super(k):
        blk = wid * n_super + k
        pltpu.sync_copy(src_hbm.at[blk], src_v)
        pltpu.sync_copy(dst_hbm.at[blk], dst_v)
        gd = [pltpu.async_copy(h_hbm.at[src_v.at[b]], rows[b], sem_g[b])
              for b in range(NBUF)]
        sd = []
        for b in range(NBUF):
            for off in win_off:
                s16 = src_v[b, pl.ds(off, L)]
                d16 = dst_v[b, pl.ds(off, L)]
                slv = plsc.load_gather(sl_v, [s16])
                srv = plsc.load_gather(sr_v, [d16])
                e_v[pl.ds(off, L)] = 1.0 / (1.0 + jnp.exp(-(slv + srv)))
            gd[b].wait()
            rb = rows[b]

            @pl.loop(0, CHUNK, unroll=4)
            def _scale(ce):
                ev = plsc.load_gather(
                    e_v, [jnp.broadcast_to(ce, (L,)).astype(jnp.int32)])
                for q in range(DIM // L):
                    rb[ce, pl.ds(q * L, L)] = rb[ce, pl.ds(q * L, L)] * ev

            sd.append(pltpu.async_copy(
                rows[b], facc.at[dst_v.at[b]], sem_s[b], add=True))
        for d in sd:
            d.wait()

    plsc.subcore_barrier()
    for kblk in range(ROWS_PER_TILE // ZBLK):
        r = row0 + kblk * ZBLK
        pltpu.sync_copy(facc.at[pl.ds(r, ZBLK)], out_hbm.at[c, pl.ds(r, ZBLK)])


@functools.lru_cache(maxsize=1)
def _sc_edge_kernel():
    mesh = plsc.VectorSubcoreMesh(
        core_axis_name="c", subcore_axis_name="s",
        num_cores=NC, num_subcores=NS)
    return pl.kernel(
        _sc_edge_body,
        out_type=jax.ShapeDtypeStruct((NC, N_PAD, DIM), jnp.float32),
        mesh=mesh,
        compiler_params=pltpu.CompilerParams(needs_layout_passes=False),
        scratch_types=[
            pltpu.VMEM((N_NODES,), jnp.float32),
            pltpu.VMEM((N_NODES,), jnp.float32),
            pltpu.VMEM((NBUF, CHUNK), jnp.int32),
            pltpu.VMEM((NBUF, CHUNK), jnp.int32),
            pltpu.VMEM((CHUNK,), jnp.float32),
            [pltpu.VMEM((CHUNK, DIM), jnp.float32)] * NBUF,
            [pltpu.SemaphoreType.DMA] * NBUF,
            [pltpu.SemaphoreType.DMA] * NBUF,
            pltpu.VMEM_SHARED((N_PAD, DIM), jnp.float32),
        ],
    )


def kernel(x, edge_index, W_fc, W_att):
    ei = edge_index.astype(jnp.int32)
    src = ei[0].reshape(-1, NBUF, CHUNK)
    dst = ei[1].reshape(-1, NBUF, CHUNK)
    h, sl, sr = pl.pallas_call(
        _tc_pre_body,
        out_shape=[
            jax.ShapeDtypeStruct((N_NODES, DIM), jnp.float32),
            jax.ShapeDtypeStruct((N_NODES,), jnp.float32),
            jax.ShapeDtypeStruct((N_NODES,), jnp.float32),
        ],
    )(x, W_fc, W_att)
    fp = _sc_edge_kernel()(h, sl, sr, src, dst)
    f = pl.pallas_call(
        _tc_add_body,
        out_shape=jax.ShapeDtypeStruct((N_NODES, DIM), jnp.float32),
    )(fp[0, :N_NODES], fp[1, :N_NODES])
    return f


# speedup vs baseline: 9.5380x; 1.0368x over previous
import functools

import jax
import jax.numpy as jnp
from jax import lax
from jax.experimental import pallas as pl
from jax.experimental.pallas import tpu as pltpu
from jax.experimental.pallas import tpu_sc as plsc

N_NODES = 10000
DIM = 128
N_EDGES = 320000

NC = 2
NS = 16
L = 16
NW = NC * NS
E_PER_W = N_EDGES // NW
CHUNK = 50
N_CHUNKS = E_PER_W // CHUNK
NBUF = 5
N_SUPER = N_CHUNKS // NBUF
N_PAD = 10112
ROWS_PER_TILE = N_PAD // NS
HIGH16 = -65536


def _tc_pre_body(x_ref, wfc_ref, watt_ref, h_ref, sl_ref, sr_ref):
    h = jnp.dot(x_ref[...], wfc_ref[...].T, preferred_element_type=jnp.float32)
    h_ref[...] = h
    al = watt_ref[0, :DIM]
    ar = watt_ref[0, DIM:]
    sl_ref[...] = jnp.sum(h * al[None, :], axis=1)
    sr_ref[...] = jnp.sum(h * ar[None, :], axis=1)


def _tc_add_body(a_ref, b_ref, o_ref):
    o_ref[...] = a_ref[...] + b_ref[...]


def _sc_edge_body(h_hbm, st_hbm, src_hbm, dst_hbm, out_hbm,
                  st_v, src_v0, src_v1, dst_v0, dst_v1, e_v, rows,
                  sem_g, sem_s, sem_is, sem_id, facc):
    srcs = (src_v0, src_v1)
    dsts = (dst_v0, dst_v1)
    c = lax.axis_index("c")
    s = lax.axis_index("s")
    wid = c * NS + s
    row0 = s * ROWS_PER_TILE

    pltpu.sync_copy(st_hbm, st_v)

    zeros = jnp.zeros((L,), jnp.float32)

    @pl.loop(0, CHUNK)
    def _zfill(i):
        for g in range(DIM // L):
            rows[0][i, pl.ds(g * L, L)] = zeros

    nz = -(-ROWS_PER_TILE // CHUNK)
    for kblk in range(nz):
        r = min(kblk * CHUNK, ROWS_PER_TILE - CHUNK)
        pltpu.sync_copy(rows[0], facc.at[pl.ds(row0 + r, CHUNK)])
    plsc.subcore_barrier()

    n_win = (CHUNK + L - 1) // L
    win_off = [min(g * L, CHUNK - L) for g in range(n_win)]

    @pl.loop(0, N_SUPER, step=2)
    def _super2(k0):
        for par in range(2):
            k = k0 + par
            sv = srcs[par]
            dv = dsts[par]
            blk = wid * N_SUPER + k
            pltpu.sync_copy(src_hbm.at[blk], sv)
            pltpu.sync_copy(dst_hbm.at[blk], dv)
            gd = [pltpu.async_copy(
                      h_hbm.at[sv.at[b]], rows[b], sem_g[b])
                  for b in range(NBUF)]
            sd = []
            for b in range(NBUF):
                for off in win_off:
                    s16 = sv[b, pl.ds(off, L)]
                    d16 = dv[b, pl.ds(off, L)]
                    w1 = plsc.load_gather(st_v, [s16])
                    w2 = plsc.load_gather(st_v, [d16])
                    slv = plsc.bitcast(w1 << 16, jnp.float32)
                    srv = plsc.bitcast(w2 & HIGH16, jnp.float32)
                    e_v[pl.ds(off, L)] = 1.0 / (1.0 + jnp.exp(-(slv + srv)))
                gd[b].wait()
                rb = rows[b]

                @pl.loop(0, CHUNK, unroll=4)
                def _scale(ce):
                    ev = plsc.load_gather(
                        e_v, [jnp.broadcast_to(ce, (L,)).astype(jnp.int32)])
                    for q in range(DIM // L):
                        rb[ce, pl.ds(q * L, L)] = rb[ce, pl.ds(q * L, L)] * ev

                sd.append(pltpu.async_copy(
                    rows[b], facc.at[dv.at[b]], sem_s[b], add=True))
            for d in sd:
                d.wait()

    plsc.subcore_barrier()
    pltpu.sync_copy(facc.at[pl.ds(row0, ROWS_PER_TILE)],
                    out_hbm.at[c, pl.ds(row0, ROWS_PER_TILE)])


@functools.lru_cache(maxsize=1)
def _sc_edge_kernel():
    mesh = plsc.VectorSubcoreMesh(
        core_axis_name="c", subcore_axis_name="s",
        num_cores=NC, num_subcores=NS)
    return pl.kernel(
        _sc_edge_body,
        out_type=jax.ShapeDtypeStruct((NC, N_PAD, DIM), jnp.float32),
        mesh=mesh,
        compiler_params=pltpu.CompilerParams(needs_layout_passes=False),
        scratch_types=[
            pltpu.VMEM((N_NODES,), jnp.int32),
            pltpu.VMEM((NBUF, CHUNK), jnp.int32),
            pltpu.VMEM((NBUF, CHUNK), jnp.int32),
            pltpu.VMEM((NBUF, CHUNK), jnp.int32),
            pltpu.VMEM((NBUF, CHUNK), jnp.int32),
            pltpu.VMEM((CHUNK,), jnp.float32),
            [pltpu.VMEM((CHUNK, DIM), jnp.float32)] * NBUF,
            [pltpu.SemaphoreType.DMA] * NBUF,
            [pltpu.SemaphoreType.DMA] * NBUF,
            [pltpu.SemaphoreType.DMA] * 2,
            [pltpu.SemaphoreType.DMA] * 2,
            pltpu.VMEM_SHARED((N_PAD, DIM), jnp.float32),
        ],
    )


def kernel(x, edge_index, W_fc, W_att):
    ei = edge_index.astype(jnp.int32)
    src = ei[0].reshape(-1, NBUF, CHUNK)
    dst = ei[1].reshape(-1, NBUF, CHUNK)
    h, sl, sr = pl.pallas_call(
        _tc_pre_body,
        out_shape=[
            jax.ShapeDtypeStruct((N_NODES, DIM), jnp.float32),
            jax.ShapeDtypeStruct((N_NODES,), jnp.float32),
            jax.ShapeDtypeStruct((N_NODES,), jnp.float32),
        ],
    )(x, W_fc, W_att)
    sl16 = lax.bitcast_convert_type(sl.astype(jnp.bfloat16), jnp.uint16)
    sr16 = lax.bitcast_convert_type(sr.astype(jnp.bfloat16), jnp.uint16)
    st = lax.bitcast_convert_type(
        sl16.astype(jnp.uint32) | (sr16.astype(jnp.uint32) << 16),
        jnp.int32)
    fp = _sc_edge_kernel()(h, st, src, dst)
    f = pl.pallas_call(
        _tc_add_body,
        out_shape=jax.ShapeDtypeStruct((N_NODES, DIM), jnp.float32),
    )(fp[0, :N_NODES], fp[1, :N_NODES])
    return f


# speedup vs baseline: 10.4067x; 1.0911x over previous
import functools

import jax
import jax.numpy as jnp
from jax import lax
from jax.experimental import pallas as pl
from jax.experimental.pallas import tpu as pltpu
from jax.experimental.pallas import tpu_sc as plsc

N_NODES = 10000
DIM = 128
N_EDGES = 320000

NC = 2
NS = 16
L = 16
NW = NC * NS
E_PER_W = N_EDGES // NW
CHUNK = 50
N_CHUNKS = E_PER_W // CHUNK
NBUF = 5
N_SUPER = N_CHUNKS // NBUF
N_PAD = 10112
ROWS_PER_TILE = N_PAD // NS
HIGH16 = -65536


def _tc_pre_body(x_ref, wfc_ref, watt_ref, h_ref, sl_ref, sr_ref):
    h = jnp.dot(x_ref[...], wfc_ref[...].T, preferred_element_type=jnp.float32)
    h_ref[...] = h
    al = watt_ref[0, :DIM]
    ar = watt_ref[0, DIM:]
    sl_ref[...] = jnp.sum(h * al[None, :], axis=1)
    sr_ref[...] = jnp.sum(h * ar[None, :], axis=1)


def _tc_add_body(a_ref, b_ref, o_ref):
    o_ref[...] = a_ref[...] + b_ref[...]


def _sc_edge_body(h_hbm, st_hbm, src_hbm, dst_hbm, out_hbm,
                  st_v, src_v0, src_v1, dst_v0, dst_v1, e_v, rows,
                  sem_g, sem_s, sem_is, sem_id, facc):
    srcs = (src_v0, src_v1)
    dsts = (dst_v0, dst_v1)
    c = lax.axis_index("c")
    s = lax.axis_index("s")
    wid = c * NS + s
    row0 = s * ROWS_PER_TILE

    pltpu.sync_copy(st_hbm, st_v)

    zeros = jnp.zeros((L,), jnp.float32)

    @pl.loop(0, CHUNK)
    def _zfill(i):
        for g in range(DIM // L):
            rows[0][i, pl.ds(g * L, L)] = zeros

    nz = -(-ROWS_PER_TILE // CHUNK)
    for kblk in range(nz):
        r = min(kblk * CHUNK, ROWS_PER_TILE - CHUNK)
        pltpu.sync_copy(rows[0], facc.at[pl.ds(row0 + r, CHUNK)])
    plsc.subcore_barrier()

    n_win = (CHUNK + L - 1) // L
    win_off = [min(g * L, CHUNK - L) for g in range(n_win)]

    @pl.loop(0, N_SUPER, step=2)
    def _super2(k0):
        for par in range(2):
            k = k0 + par
            sv = srcs[par]
            dv = dsts[par]
            blk = wid * N_SUPER + k
            pltpu.sync_copy(src_hbm.at[blk], sv)
            pltpu.sync_copy(dst_hbm.at[blk], dv)
            gd = [pltpu.async_copy(
                      h_hbm.at[sv.at[b]], rows[b], sem_g[b])
                  for b in range(NBUF)]
            sd = []
            for b in range(NBUF):
                for off in win_off:
                    s16 = sv[b, pl.ds(off, L)]
                    d16 = dv[b, pl.ds(off, L)]
                    w1 = plsc.load_gather(st_v, [s16])
                    w2 = plsc.load_gather(st_v, [d16])
                    slv = plsc.bitcast(w1 << 16, jnp.float32)
                    srv = plsc.bitcast(w2 & HIGH16, jnp.float32)
                    e_v[pl.ds(off, L)] = 1.0 / (1.0 + jnp.exp(-(slv + srv)))
                gd[b].wait()
                rb = rows[b]

                @plsc.parallel_loop(0, CHUNK, unroll=4)
                def _scale(ce):
                    ev = plsc.load_gather(
                        e_v, [jnp.broadcast_to(ce, (L,)).astype(jnp.int32)])
                    for q in range(DIM // L):
                        rb[ce, pl.ds(q * L, L)] = rb[ce, pl.ds(q * L, L)] * ev

                sd.append(pltpu.async_copy(
                    rows[b], facc.at[dv.at[b]], sem_s[b], add=True))
            for d in sd:
                d.wait()

    plsc.subcore_barrier()
    pltpu.sync_copy(facc.at[pl.ds(row0, ROWS_PER_TILE)],
                    out_hbm.at[c, pl.ds(row0, ROWS_PER_TILE)])


@functools.lru_cache(maxsize=1)
def _sc_edge_kernel():
    mesh = plsc.VectorSubcoreMesh(
        core_axis_name="c", subcore_axis_name="s",
        num_cores=NC, num_subcores=NS)
    return pl.kernel(
        _sc_edge_body,
        out_type=jax.ShapeDtypeStruct((NC, N_PAD, DIM), jnp.float32),
        mesh=mesh,
        compiler_params=pltpu.CompilerParams(needs_layout_passes=False),
        scratch_types=[
            pltpu.VMEM((N_NODES,), jnp.int32),
            pltpu.VMEM((NBUF, CHUNK), jnp.int32),
            pltpu.VMEM((NBUF, CHUNK), jnp.int32),
            pltpu.VMEM((NBUF, CHUNK), jnp.int32),
            pltpu.VMEM((NBUF, CHUNK), jnp.int32),
            pltpu.VMEM((CHUNK,), jnp.float32),
            [pltpu.VMEM((CHUNK, DIM), jnp.float32)] * NBUF,
            [pltpu.SemaphoreType.DMA] * NBUF,
            [pltpu.SemaphoreType.DMA] * NBUF,
            [pltpu.SemaphoreType.DMA] * 2,
            [pltpu.SemaphoreType.DMA] * 2,
            pltpu.VMEM_SHARED((N_PAD, DIM), jnp.float32),
        ],
    )


def kernel(x, edge_index, W_fc, W_att):
    ei = edge_index.astype(jnp.int32)
    src = ei[0].reshape(-1, NBUF, CHUNK)
    dst = ei[1].reshape(-1, NBUF, CHUNK)
    h, sl, sr = pl.pallas_call(
        _tc_pre_body,
        out_shape=[
            jax.ShapeDtypeStruct((N_NODES, DIM), jnp.float32),
            jax.ShapeDtypeStruct((N_NODES,), jnp.float32),
            jax.ShapeDtypeStruct((N_NODES,), jnp.float32),
        ],
    )(x, W_fc, W_att)
    sl16 = lax.bitcast_convert_type(sl.astype(jnp.bfloat16), jnp.uint16)
    sr16 = lax.bitcast_convert_type(sr.astype(jnp.bfloat16), jnp.uint16)
    st = lax.bitcast_convert_type(
        sl16.astype(jnp.uint32) | (sr16.astype(jnp.uint32) << 16),
        jnp.int32)
    fp = _sc_edge_kernel()(h, st, src, dst)
    f = pl.pallas_call(
        _tc_add_body,
        out_shape=jax.ShapeDtypeStruct((N_NODES, DIM), jnp.float32),
    )(fp[0, :N_NODES], fp[1, :N_NODES])
    return f


# speedup vs baseline: 12.0218x; 1.1552x over previous
import functools

import jax
import jax.numpy as jnp
from jax import lax
from jax.experimental import pallas as pl
from jax.experimental.pallas import tpu as pltpu
from jax.experimental.pallas import tpu_sc as plsc

N_NODES = 10000
DIM = 128
N_EDGES = 320000

NC = 2
NS = 16
L = 16
NW = NC * NS
E_PER_W = N_EDGES // NW
CHUNK = 50
N_CHUNKS = E_PER_W // CHUNK
NBUF = 5
N_SUPER = N_CHUNKS // NBUF
N_PAD = 10112
ROWS_PER_TILE = N_PAD // NS
HIGH16 = -65536


def _tc_pre_body(x_ref, wfc_ref, watt_ref, h_ref, sl_ref, sr_ref):
    h = jnp.dot(x_ref[...], wfc_ref[...].T, preferred_element_type=jnp.float32)
    h_ref[...] = h
    al = watt_ref[0, :DIM]
    ar = watt_ref[0, DIM:]
    sl_ref[...] = jnp.sum(h * al[None, :], axis=1)
    sr_ref[...] = jnp.sum(h * ar[None, :], axis=1)


def _tc_add_body(a_ref, b_ref, o_ref):
    o_ref[...] = a_ref[...] + b_ref[...]


def _sc_edge_body(h_hbm, st_hbm, src_hbm, dst_hbm, out_hbm,
                  st_v, src_v0, src_v1, dst_v0, dst_v1, e_v, rows,
                  sem_g, sem_s, sem_is, sem_id, facc):
    srcs = (src_v0, src_v1)
    dsts = (dst_v0, dst_v1)
    c = lax.axis_index("c")
    s = lax.axis_index("s")
    wid = c * NS + s
    row0 = s * ROWS_PER_TILE

    pltpu.sync_copy(st_hbm, st_v)

    zeros = jnp.zeros((L,), jnp.float32)

    @pl.loop(0, CHUNK)
    def _zfill(i):
        for g in range(DIM // L):
            rows[0][i, pl.ds(g * L, L)] = zeros

    nz = -(-ROWS_PER_TILE // CHUNK)
    for kblk in range(nz):
        r = min(kblk * CHUNK, ROWS_PER_TILE - CHUNK)
        pltpu.sync_copy(rows[0], facc.at[pl.ds(row0 + r, CHUNK)])
    plsc.subcore_barrier()

    n_win = (CHUNK + L - 1) // L
    win_off = [min(g * L, CHUNK - L) for g in range(n_win)]

    def _idx_load(par, k):
        blk = wid * N_SUPER + k
        pltpu.async_copy(src_hbm.at[blk], srcs[par], sem_is[par])
        pltpu.async_copy(dst_hbm.at[blk], dsts[par], sem_id[par])

    _idx_load(0, 0)
    _idx_load(1, 1)

    @pl.loop(0, N_SUPER, step=2)
    def _super2(k0):
        for par in range(2):
            k = k0 + par
            sv = srcs[par]
            dv = dsts[par]
            pltpu.make_async_copy(
                src_hbm.at[wid * N_SUPER], sv, sem_is[par]).wait()
            pltpu.make_async_copy(
                dst_hbm.at[wid * N_SUPER], dv, sem_id[par]).wait()
            gd = [pltpu.async_copy(
                      h_hbm.at[sv.at[b]], rows[b], sem_g[b])
                  for b in range(NBUF)]
            sd = []
            for b in range(NBUF):
                for off in win_off:
                    s16 = sv[b, pl.ds(off, L)]
                    d16 = dv[b, pl.ds(off, L)]
                    w1 = plsc.load_gather(st_v, [s16])
                    w2 = plsc.load_gather(st_v, [d16])
                    slv = plsc.bitcast(w1 << 16, jnp.float32)
                    srv = plsc.bitcast(w2 & HIGH16, jnp.float32)
                    e_v[pl.ds(off, L)] = 1.0 / (1.0 + jnp.exp(-(slv + srv)))
                gd[b].wait()
                rb = rows[b]

                @plsc.parallel_loop(0, CHUNK, unroll=4)
                def _scale(ce):
                    ev = plsc.load_gather(
                        e_v, [jnp.broadcast_to(ce, (L,)).astype(jnp.int32)])
                    for q in range(DIM // L):
                        rb[ce, pl.ds(q * L, L)] = rb[ce, pl.ds(q * L, L)] * ev

                sd.append(pltpu.async_copy(
                    rows[b], facc.at[dv.at[b]], sem_s[b], add=True))
            for d in sd:
                d.wait()

            @pl.when(k + 2 < N_SUPER)
            def _prefetch():
                _idx_load(par, k + 2)

    plsc.subcore_barrier()
    pltpu.sync_copy(facc.at[pl.ds(row0, ROWS_PER_TILE)],
                    out_hbm.at[c, pl.ds(row0, ROWS_PER_TILE)])


@functools.lru_cache(maxsize=1)
def _sc_edge_kernel():
    mesh = plsc.VectorSubcoreMesh(
        core_axis_name="c", subcore_axis_name="s",
        num_cores=NC, num_subcores=NS)
    return pl.kernel(
        _sc_edge_body,
        out_type=jax.ShapeDtypeStruct((NC, N_PAD, DIM), jnp.float32),
        mesh=mesh,
        compiler_params=pltpu.CompilerParams(needs_layout_passes=False),
        scratch_types=[
            pltpu.VMEM((N_NODES,), jnp.int32),
            pltpu.VMEM((NBUF, CHUNK), jnp.int32),
            pltpu.VMEM((NBUF, CHUNK), jnp.int32),
            pltpu.VMEM((NBUF, CHUNK), jnp.int32),
            pltpu.VMEM((NBUF, CHUNK), jnp.int32),
            pltpu.VMEM((CHUNK,), jnp.float32),
            [pltpu.VMEM((CHUNK, DIM), jnp.float32)] * NBUF,
            [pltpu.SemaphoreType.DMA] * NBUF,
            [pltpu.SemaphoreType.DMA] * NBUF,
            [pltpu.SemaphoreType.DMA] * 2,
            [pltpu.SemaphoreType.DMA] * 2,
            pltpu.VMEM_SHARED((N_PAD, DIM), jnp.float32),
        ],
    )


def kernel(x, edge_index, W_fc, W_att):
    ei = edge_index.astype(jnp.int32)
    src = ei[0].reshape(-1, NBUF, CHUNK)
    dst = ei[1].reshape(-1, NBUF, CHUNK)
    h, sl, sr = pl.pallas_call(
        _tc_pre_body,
        out_shape=[
            jax.ShapeDtypeStruct((N_NODES, DIM), jnp.float32),
            jax.ShapeDtypeStruct((N_NODES,), jnp.float32),
            jax.ShapeDtypeStruct((N_NODES,), jnp.float32),
        ],
    )(x, W_fc, W_att)
    sl16 = lax.bitcast_convert_type(sl.astype(jnp.bfloat16), jnp.uint16)
    sr16 = lax.bitcast_convert_type(sr.astype(jnp.bfloat16), jnp.uint16)
    st = lax.bitcast_convert_type(
        sl16.astype(jnp.uint32) | (sr16.astype(jnp.uint32) << 16),
        jnp.int32)
    fp = _sc_edge_kernel()(h, st, src, dst)
    f = pl.pallas_call(
        _tc_add_body,
        out_shape=jax.ShapeDtypeStruct((N_NODES, DIM), jnp.float32),
    )(fp[0, :N_NODES], fp[1, :N_NODES])
    return f


# speedup vs baseline: 13.4272x; 1.1169x over previous
import functools

import jax
import jax.numpy as jnp
from jax import lax
from jax.experimental import pallas as pl
from jax.experimental.pallas import tpu as pltpu
from jax.experimental.pallas import tpu_sc as plsc

N_NODES = 10000
DIM = 128
N_EDGES = 320000

NC = 2
NS = 16
L = 16
NW = NC * NS
E_PER_W = N_EDGES // NW
CHUNK = 50
N_CHUNKS = E_PER_W // CHUNK
NBUF = 5
N_SUPER = N_CHUNKS // NBUF
N_PAD = 10112
ROWS_PER_TILE = N_PAD // NS
HIGH16 = -65536


def _tc_pre_body(x_ref, wfc_ref, watt_ref, h_ref, sl_ref, sr_ref):
    h = jnp.dot(x_ref[...], wfc_ref[...].T, preferred_element_type=jnp.float32)
    h_ref[...] = h
    al = watt_ref[0, :DIM]
    ar = watt_ref[0, DIM:]
    sl_ref[...] = jnp.sum(h * al[None, :], axis=1)
    sr_ref[...] = jnp.sum(h * ar[None, :], axis=1)


def _tc_add_body(fp_ref, o_ref):
    o_ref[...] = fp_ref[0, :N_NODES, :] + fp_ref[1, :N_NODES, :]


def _sc_edge_body(h_hbm, st_hbm, edge_hbm, out_hbm,
                  st_v, sv, dv, e_v, rows,
                  sem_g, sem_s, sem_is, sem_id, facc):
    c = lax.axis_index("c")
    s = lax.axis_index("s")
    wid = c * NS + s
    row0 = s * ROWS_PER_TILE

    pltpu.sync_copy(st_hbm, st_v)

    zeros = jnp.zeros((L,), jnp.float32)

    @pl.loop(0, CHUNK)
    def _zfill(i):
        for g in range(DIM // L):
            rows[0][i, pl.ds(g * L, L)] = zeros

    nz = -(-ROWS_PER_TILE // CHUNK)
    zd = []
    for kblk in range(nz):
        r = min(kblk * CHUNK, ROWS_PER_TILE - CHUNK)
        zd.append(pltpu.async_copy(
            rows[0], facc.at[pl.ds(row0 + r, CHUNK)], sem_g[0]))
    for d in zd:
        d.wait()
    plsc.subcore_barrier()

    n_win = (CHUNK + L - 1) // L
    win_off = [min(g * L, CHUNK - L) for g in range(n_win)]

    def _idx_load(k):
        blk = wid * N_SUPER + k
        pltpu.async_copy(edge_hbm.at[0, blk], sv, sem_is)
        pltpu.async_copy(edge_hbm.at[1, blk], dv, sem_id)

    _idx_load(0)

    @pl.loop(0, N_SUPER)
    def _super(k):
        if True:
            pltpu.make_async_copy(
                edge_hbm.at[0, wid * N_SUPER], sv, sem_is).wait()
            pltpu.make_async_copy(
                edge_hbm.at[1, wid * N_SUPER], dv, sem_id).wait()
            gd = [pltpu.async_copy(
                      h_hbm.at[sv.at[b]], rows[b], sem_g[b])
                  for b in range(NBUF)]
            for b in range(NBUF):
                for off in win_off:
                    s16 = sv[b, pl.ds(off, L)]
                    d16 = dv[b, pl.ds(off, L)]
                    w1 = plsc.load_gather(st_v, [s16])
                    w2 = plsc.load_gather(st_v, [d16])
                    slv = plsc.bitcast(w1 << 16, jnp.float32)
                    srv = plsc.bitcast(w2 & HIGH16, jnp.float32)
                    e_v[pl.ds(b * CHUNK + off, L)] = (
                        1.0 / (1.0 + jnp.exp(-(slv + srv))))
            sd = []
            for b in range(NBUF):
                gd[b].wait()
                rb = rows[b]
                eb = b * CHUNK

                @plsc.parallel_loop(0, CHUNK, unroll=4)
                def _scale(ce):
                    ev = plsc.load_gather(
                        e_v,
                        [jnp.broadcast_to(eb + ce, (L,)).astype(jnp.int32)])
                    for q in range(DIM // L):
                        rb[ce, pl.ds(q * L, L)] = rb[ce, pl.ds(q * L, L)] * ev

                sd.append(pltpu.async_copy(
                    rows[b], facc.at[dv.at[b]], sem_s[b], add=True))
            for d in sd[:-1]:
                d.wait()

            @pl.when(k + 1 < N_SUPER)
            def _prefetch():
                _idx_load(k + 1)

            sd[-1].wait()

    plsc.subcore_barrier()
    pltpu.sync_copy(facc.at[pl.ds(row0, ROWS_PER_TILE)],
                    out_hbm.at[c, pl.ds(row0, ROWS_PER_TILE)])


@functools.lru_cache(maxsize=1)
def _sc_edge_kernel():
    mesh = plsc.VectorSubcoreMesh(
        core_axis_name="c", subcore_axis_name="s",
        num_cores=NC, num_subcores=NS)
    return pl.kernel(
        _sc_edge_body,
        out_type=jax.ShapeDtypeStruct((NC, N_PAD, DIM), jnp.float32),
        mesh=mesh,
        compiler_params=pltpu.CompilerParams(needs_layout_passes=False),
        scratch_types=[
            pltpu.VMEM((N_NODES,), jnp.int32),
            pltpu.VMEM((NBUF, CHUNK), jnp.int32),
            pltpu.VMEM((NBUF, CHUNK), jnp.int32),
            pltpu.VMEM((NBUF * CHUNK,), jnp.float32),
            [pltpu.VMEM((CHUNK, DIM), jnp.float32)] * NBUF,
            [pltpu.SemaphoreType.DMA] * NBUF,
            [pltpu.SemaphoreType.DMA] * NBUF,
            pltpu.SemaphoreType.DMA,
            pltpu.SemaphoreType.DMA,
            pltpu.VMEM_SHARED((N_PAD, DIM), jnp.float32),
        ],
    )


def kernel(x, edge_index, W_fc, W_att):
    e4 = edge_index.astype(jnp.int32).reshape(2, -1, NBUF, CHUNK)
    h, sl, sr = pl.pallas_call(
        _tc_pre_body,
        out_shape=[
            jax.ShapeDtypeStruct((N_NODES, DIM), jnp.float32),
            jax.ShapeDtypeStruct((N_NODES,), jnp.float32),
            jax.ShapeDtypeStruct((N_NODES,), jnp.float32),
        ],
    )(x, W_fc, W_att)
    sl16 = lax.bitcast_convert_type(sl.astype(jnp.bfloat16), jnp.uint16)
    sr16 = lax.bitcast_convert_type(sr.astype(jnp.bfloat16), jnp.uint16)
    st = lax.bitcast_convert_type(
        sl16.astype(jnp.uint32) | (sr16.astype(jnp.uint32) << 16),
        jnp.int32)
    fp = _sc_edge_kernel()(h, st, e4)
    f = pl.pallas_call(
        _tc_add_body,
        out_shape=jax.ShapeDtypeStruct((N_NODES, DIM), jnp.float32),
    )(fp)
    return f


# speedup vs baseline: 13.6394x; 1.0158x over previous
import functools

import jax
import jax.numpy as jnp
from jax import lax
from jax.experimental import pallas as pl
from jax.experimental.pallas import tpu as pltpu
from jax.experimental.pallas import tpu_sc as plsc

N_NODES = 10000
DIM = 128
N_EDGES = 320000

NC = 2
NS = 16
L = 16
NW = NC * NS
E_PER_W = N_EDGES // NW
CHUNK = 50
N_CHUNKS = E_PER_W // CHUNK
NBUF = 5
N_SUPER = N_CHUNKS // NBUF
N_PAD = 10112
ROWS_PER_TILE = N_PAD // NS
HIGH16 = -65536


BLK = 1000


def _tc_pre_body(x_ref, wfc_ref, watt_ref, h_ref, st_ref):
    h = jnp.dot(x_ref[...], wfc_ref[...].T, preferred_element_type=jnp.float32)
    h_ref[...] = h
    al = watt_ref[0, :DIM]
    ar = watt_ref[0, DIM:]
    sl = jnp.sum(h * al[None, :], axis=1)
    sr = jnp.sum(h * ar[None, :], axis=1)
    slb = lax.bitcast_convert_type(
        sl.astype(jnp.bfloat16).astype(jnp.float32), jnp.uint32)
    srb = lax.bitcast_convert_type(
        sr.astype(jnp.bfloat16).astype(jnp.float32), jnp.uint32)
    st_ref[...] = lax.bitcast_convert_type(
        (slb >> 16) | (srb & jnp.uint32(0xFFFF0000)), jnp.int32)


def _tc_add_body(fp_ref, o_ref):
    o_ref[...] = fp_ref[0, :N_NODES, :] + fp_ref[1, :N_NODES, :]


def _sc_edge_body(h_hbm, st_hbm, edge_hbm, out_hbm,
                  st_v, sv, dv, e_v, rows,
                  sem_g, sem_s, sem_is, sem_id, facc):
    c = lax.axis_index("c")
    s = lax.axis_index("s")
    wid = c * NS + s
    row0 = s * ROWS_PER_TILE

    pltpu.sync_copy(st_hbm, st_v)

    zeros = jnp.zeros((L,), jnp.float32)

    @pl.loop(0, CHUNK)
    def _zfill(i):
        for g in range(DIM // L):
            rows[0][i, pl.ds(g * L, L)] = zeros

    nz = -(-ROWS_PER_TILE // CHUNK)
    zd = []
    for kblk in range(nz):
        r = min(kblk * CHUNK, ROWS_PER_TILE - CHUNK)
        zd.append(pltpu.async_copy(
            rows[0], facc.at[pl.ds(row0 + r, CHUNK)], sem_g[0]))
    for d in zd:
        d.wait()
    plsc.subcore_barrier()

    n_win = (CHUNK + L - 1) // L
    win_off = [min(g * L, CHUNK - L) for g in range(n_win)]

    def _idx_load(k):
        blk = wid * N_SUPER + k
        pltpu.async_copy(edge_hbm.at[0, blk], sv, sem_is)
        pltpu.async_copy(edge_hbm.at[1, blk], dv, sem_id)

    _idx_load(0)

    @pl.loop(0, N_SUPER)
    def _super(k):
        if True:
            pltpu.make_async_copy(
                edge_hbm.at[0, wid * N_SUPER], sv, sem_is).wait()
            pltpu.make_async_copy(
                edge_hbm.at[1, wid * N_SUPER], dv, sem_id).wait()
            gd = [pltpu.async_copy(
                      h_hbm.at[sv.at[b]], rows[b], sem_g[b])
                  for b in range(NBUF)]
            for b in range(NBUF):
                for off in win_off:
                    s16 = sv[b, pl.ds(off, L)]
                    d16 = dv[b, pl.ds(off, L)]
                    w1 = plsc.load_gather(st_v, [s16])
                    w2 = plsc.load_gather(st_v, [d16])
                    slv = plsc.bitcast(w1 << 16, jnp.float32)
                    srv = plsc.bitcast(w2 & HIGH16, jnp.float32)
                    e_v[pl.ds(b * CHUNK + off, L)] = (
                        1.0 / (1.0 + jnp.exp(-(slv + srv))))
            sd = []
            for b in range(NBUF):
                gd[b].wait()
                rb = rows[b]
                eb = b * CHUNK

                @plsc.parallel_loop(0, CHUNK, unroll=4)
                def _scale(ce):
                    ev = plsc.load_gather(
                        e_v,
                        [jnp.broadcast_to(eb + ce, (L,)).astype(jnp.int32)])
                    for q in range(DIM // L):
                        rb[ce, pl.ds(q * L, L)] = rb[ce, pl.ds(q * L, L)] * ev

                sd.append(pltpu.async_copy(
                    rows[b], facc.at[dv.at[b]], sem_s[b], add=True))
            for d in sd[:-1]:
                d.wait()

            @pl.when(k + 1 < N_SUPER)
            def _prefetch():
                _idx_load(k + 1)

            sd[-1].wait()

    plsc.subcore_barrier()
    pltpu.sync_copy(facc.at[pl.ds(row0, ROWS_PER_TILE)],
                    out_hbm.at[c, pl.ds(row0, ROWS_PER_TILE)])


@functools.lru_cache(maxsize=1)
def _sc_edge_kernel():
    mesh = plsc.VectorSubcoreMesh(
        core_axis_name="c", subcore_axis_name="s",
        num_cores=NC, num_subcores=NS)
    return pl.kernel(
        _sc_edge_body,
        out_type=jax.ShapeDtypeStruct((NC, N_PAD, DIM), jnp.float32),
        mesh=mesh,
        compiler_params=pltpu.CompilerParams(needs_layout_passes=False),
        scratch_types=[
            pltpu.VMEM((N_NODES,), jnp.int32),
            pltpu.VMEM((NBUF, CHUNK), jnp.int32),
            pltpu.VMEM((NBUF, CHUNK), jnp.int32),
            pltpu.VMEM((NBUF * CHUNK,), jnp.float32),
            [pltpu.VMEM((CHUNK, DIM), jnp.float32)] * NBUF,
            [pltpu.SemaphoreType.DMA] * NBUF,
            [pltpu.SemaphoreType.DMA] * NBUF,
            pltpu.SemaphoreType.DMA,
            pltpu.SemaphoreType.DMA,
            pltpu.VMEM_SHARED((N_PAD, DIM), jnp.float32),
        ],
    )


def kernel(x, edge_index, W_fc, W_att):
    e4 = edge_index.astype(jnp.int32).reshape(2, -1, NBUF, CHUNK)
    h, st = pl.pallas_call(
        _tc_pre_body,
        out_shape=[
            jax.ShapeDtypeStruct((N_NODES, DIM), jnp.float32),
            jax.ShapeDtypeStruct((N_NODES,), jnp.int32),
        ],
    )(x, W_fc, W_att)
    fp = _sc_edge_kernel()(h, st, e4)
    f = pl.pallas_call(
        _tc_add_body,
        out_shape=jax.ShapeDtypeStruct((N_NODES, DIM), jnp.float32),
    )(fp)
    return f
